# Initial kernel scaffold; baseline (speedup 1.0000x reference)
import jax
import jax.numpy as jnp
from jax.experimental import pallas as pl


def kernel(hidden_states, instrument_ids, session_ids, instrument_table, session_table):
    raise NotImplementedError("write your pallas kernel here")


# speedup vs baseline: 1.9799x; 1.9799x over previous
import jax
import jax.numpy as jnp
from jax import lax
from jax.experimental import pallas as pl
from jax.experimental.pallas import tpu as pltpu
from jax.experimental.pallas import tpu_sc as plsc

NC = 2
NS = 16
NW = NC * NS

B, S, H = 4096, 200, 64
BPW = B // NW
S0 = 128
S1 = S - S0


def _body(hid_hbm, iid_hbm, sid_hbm, itab_hbm, stab_hbm, out_hbm,
          inst_idx_v, inst_rows_v, sidx_v, sess_rows_v, hid_v, sem, gsem):
    cid = lax.axis_index("c")
    sid = lax.axis_index("s")
    wid = sid * NC + cid
    base = wid * BPW

    pltpu.sync_copy(iid_hbm.at[pl.ds(base, BPW)], inst_idx_v)
    pltpu.async_copy(itab_hbm.at[inst_idx_v], inst_rows_v, gsem).wait()

    def row(r, carry):
        b = base + r
        pltpu.sync_copy(sid_hbm.at[b, pl.ds(0, S0)], sidx_v.at[0])
        pltpu.sync_copy(sid_hbm.at[b, pl.ds(S0, S1)], sidx_v.at[1, pl.ds(0, S1)])
        cp_h = pltpu.async_copy(hid_hbm.at[b], hid_v, sem)
        cp_a = pltpu.async_copy(stab_hbm.at[sidx_v.at[0]],
                                sess_rows_v.at[pl.ds(0, S0)], gsem)
        cp_b = pltpu.async_copy(stab_hbm.at[sidx_v.at[1, pl.ds(0, S1)]],
                                sess_rows_v.at[pl.ds(S0, S1)], gsem)
        cp_h.wait()
        cp_a.wait()
        cp_b.wait()

        inst = [inst_rows_v[r, pl.ds(16 * j, 16)] for j in range(4)]

        def pos(s, c):
            for j in range(4):
                hid_v[s, pl.ds(16 * j, 16)] = (
                    hid_v[s, pl.ds(16 * j, 16)]
                    + sess_rows_v[s, pl.ds(16 * j, 16)]
                    + inst[j])
            return c

        lax.fori_loop(0, S, pos, 0)
        pltpu.sync_copy(hid_v, out_hbm.at[b])
        return carry

    lax.fori_loop(0, BPW, row, 0)


def kernel(hidden_states, instrument_ids, session_ids, instrument_table,
           session_table):
    k = pl.kernel(
        _body,
        out_type=jax.ShapeDtypeStruct((B, S, H), jnp.float32),
        mesh=plsc.VectorSubcoreMesh(core_axis_name="c", subcore_axis_name="s",
                                    num_cores=NC, num_subcores=NS),
        compiler_params=pltpu.CompilerParams(use_tc_tiling_on_sc=False),
        scratch_types=[
            pltpu.VMEM((BPW,), jnp.int32),
            pltpu.VMEM((BPW, H), jnp.float32),
            pltpu.VMEM((2, 128), jnp.int32),
            pltpu.VMEM((S, H), jnp.float32),
            pltpu.VMEM((S, H), jnp.float32),
            pltpu.SemaphoreType.DMA,
            pltpu.SemaphoreType.DMA,
        ],
    )
    return k(hidden_states, instrument_ids.astype(jnp.int32),
             session_ids.astype(jnp.int32), instrument_table, session_table)


# speedup vs baseline: 2.3696x; 1.1969x over previous
import jax
import jax.numpy as jnp
from jax import lax
from jax.experimental import pallas as pl
from jax.experimental.pallas import tpu as pltpu
from jax.experimental.pallas import tpu_sc as plsc

NC = 2
NS = 16
NW = NC * NS

B, S, H = 4096, 200, 64
BPW = B // NW
S0 = 128
S1 = S - S0
NBUF = 3
NGRP = BPW // NBUF


def _body(hid_hbm, iid_hbm, sid_hbm, itab_hbm, stab_hbm, out_hbm,
          ids_v, inst_idx_v, inst_rows_v,
          hid0, hid1, hid2, sess0, sess1, sess2,
          si0, si1, si2, so0, so1, so2, gsem):
    hid_bufs = (hid0, hid1, hid2)
    sess_bufs = (sess0, sess1, sess2)
    sem_in = (si0, si1, si2)
    sem_out = (so0, so1, so2)

    cid = lax.axis_index("c")
    sid = lax.axis_index("s")
    wid = sid * NC + cid
    base = wid * BPW

    pltpu.sync_copy(sid_hbm.at[pl.ds(base * S, BPW * S)], ids_v)
    pltpu.sync_copy(iid_hbm.at[pl.ds(base, BPW)], inst_idx_v)
    pltpu.async_copy(itab_hbm.at[inst_idx_v], inst_rows_v, gsem).wait()

    def in_copies(r, k):
        return (
            pltpu.make_async_copy(hid_hbm.at[base + r], hid_bufs[k], sem_in[k]),
            pltpu.make_async_copy(stab_hbm.at[ids_v.at[pl.ds(r * S, S0)]],
                                  sess_bufs[k].at[pl.ds(0, S0)], sem_in[k]),
            pltpu.make_async_copy(stab_hbm.at[ids_v.at[pl.ds(r * S + S0, S1)]],
                                  sess_bufs[k].at[pl.ds(S0, S1)], sem_in[k]),
        )

    def out_copy(r, k):
        return pltpu.make_async_copy(hid_bufs[k], out_hbm.at[base + r],
                                     sem_out[k])

    def fire_in(r, k):
        for c in in_copies(r, k):
            c.start()

    def wait_in(r, k):
        for c in in_copies(r, k):
            c.wait()

    def compute(r, k):
        hid_b = hid_bufs[k]
        sess_b = sess_bufs[k]
        inst = [inst_rows_v[r, pl.ds(16 * j, 16)] for j in range(4)]

        def pos(s, c):
            for j in range(4):
                plsc.addupdate(hid_b.at[s, pl.ds(16 * j, 16)],
                               sess_b[s, pl.ds(16 * j, 16)] + inst[j])
            return c

        lax.fori_loop(0, S, pos, 0, unroll=2)

    fire_in(0, 0)
    fire_in(1, 1)

    wait_in(0, 0)
    compute(0, 0)
    out_copy(0, 0).start()
    fire_in(2, 2)

    wait_in(1, 1)
    compute(1, 1)
    out_copy(1, 1).start()
    out_copy(0, 0).wait()
    fire_in(3, 0)

    wait_in(2, 2)
    compute(2, 2)
    out_copy(2, 2).start()
    out_copy(1, 1).wait()
    fire_in(4, 1)

    def group(g, carry):
        for b in range(NBUF):
            r = NBUF * g + b
            k = b
            k2 = (b + 2) % NBUF
            wait_in(r, k)
            compute(r, k)
            out_copy(r, k).start()
            out_copy(r - 1, k2).wait()
            fire_in(r + 2, k2)
        return carry

    lax.fori_loop(1, NGRP, group, 0)

    r = NBUF * NGRP
    wait_in(r, 0)
    compute(r, 0)
    out_copy(r, 0).start()

    wait_in(r + 1, 1)
    compute(r + 1, 1)
    out_copy(r + 1, 1).start()

    out_copy(r - 1, 2).wait()
    out_copy(r, 0).wait()
    out_copy(r + 1, 1).wait()


def kernel(hidden_states, instrument_ids, session_ids, instrument_table,
           session_table):
    k = pl.kernel(
        _body,
        out_type=jax.ShapeDtypeStruct((B, S, H), jnp.float32),
        mesh=plsc.VectorSubcoreMesh(core_axis_name="c", subcore_axis_name="s",
                                    num_cores=NC, num_subcores=NS),
        compiler_params=pltpu.CompilerParams(use_tc_tiling_on_sc=False),
        scratch_types=(
            [pltpu.VMEM((BPW * S,), jnp.int32),
             pltpu.VMEM((BPW,), jnp.int32),
             pltpu.VMEM((BPW, H), jnp.float32)]
            + [pltpu.VMEM((S, H), jnp.float32) for _ in range(2 * NBUF)]
            + [pltpu.SemaphoreType.DMA for _ in range(2 * NBUF + 1)]
        ),
    )
    return k(hidden_states, instrument_ids.astype(jnp.int32),
             session_ids.reshape(-1).astype(jnp.int32),
             instrument_table, session_table)
